# baseline (speedup 1.0000x reference)
import jax
import jax.numpy as jnp
from jax import lax
from jax.experimental import pallas as pl
from jax.experimental.pallas import tpu as pltpu
from jax.experimental.pallas import tpu_sc as plsc

NUM_ENT = 1000000
NUM_REL = 1000
DIM = 32
BATCH = 16384

_INFO = plsc.get_sparse_core_info()
NC = _INFO.num_cores
NS = _INFO.num_subcores
NW = NC * NS
B_PER_W = BATCH // NW
G = 16
NG = B_PER_W // G
NBLK = NUM_ENT // 4


def _newton_sqrt(x):
  bits = plsc.bitcast(x, jnp.int32)
  seed = jnp.int32(0x5F3759DF) - lax.shift_right_logical(bits, 1)
  y = plsc.bitcast(seed, jnp.float32)
  half = x * 0.5
  for _ in range(3):
    y = y * (1.5 - half * y * y)
  return x * y


def _body(eh3, rvh_flat, u1, r1, v1, out, uidx, ridx, vidx,
          rbuf, ku, kv, du, dv, ov, sems, rsem):
  wid = lax.axis_index("s") * NC + lax.axis_index("c")
  base = wid * B_PER_W

  rcp = pltpu.async_copy(rvh_flat, rbuf, rsem)
  pltpu.sync_copy(u1.at[pl.ds(base, B_PER_W)], uidx)
  pltpu.sync_copy(v1.at[pl.ds(base, B_PER_W)], vidx)
  pltpu.sync_copy(r1.at[pl.ds(base, B_PER_W)], ridx)

  iota = lax.iota(jnp.int32, 16)

  def _fire(idxv, i0, kst, dst, sem):
    ivec = idxv[pl.ds(i0, G)]
    kst[...] = lax.shift_right_logical(ivec, 2)
    return pltpu.async_copy(eh3.at[kst], dst, sem)

  for g in (0, 1):
    _fire(uidx, g * G, ku.at[g], du.at[g], sems.at[g])
    _fire(vidx, g * G, kv.at[g], dv.at[g], sems.at[g])

  rcp.wait()

  @pl.loop(0, NG)
  def _group(g):
    slot = lax.rem(g, 2)
    i0 = g * G
    pltpu.make_async_copy(eh3.at[ku.at[slot]], du.at[slot],
                          sems.at[slot]).wait()
    pltpu.make_async_copy(eh3.at[kv.at[slot]], dv.at[slot],
                          sems.at[slot]).wait()

    ivu = uidx[pl.ds(i0, G)]
    ivv = vidx[pl.ds(i0, G)]
    ivr = ridx[pl.ds(i0, G)]
    su = (ivu & jnp.int32(3)) * jnp.int32(DIM)
    sv = (ivv & jnp.int32(3)) * jnp.int32(DIM)
    rofs = ivr * jnp.int32(DIM)
    dus = du.at[slot]
    dvs = dv.at[slot]
    acc = jnp.zeros((16,), jnp.float32)
    for d in range(DIM):
      dd = jnp.full((16,), d, jnp.int32)
      uval = plsc.load_gather(dus, [iota, su + dd])
      vval = plsc.load_gather(dvs, [iota, sv + dd])
      rval = plsc.load_gather(rbuf, [rofs + jnp.int32(d)])
      diff = uval + rval - vval
      acc = acc + diff * diff
    ov[pl.ds(i0, G)] = _newton_sqrt(acc)

    @pl.when(g < NG - 2)
    def _refill():
      i2 = i0 + 2 * G
      _fire(uidx, i2, ku.at[slot], du.at[slot], sems.at[slot])
      _fire(vidx, i2, kv.at[slot], dv.at[slot], sems.at[slot])

  pltpu.sync_copy(ov, out.at[pl.ds(base, B_PER_W)])


@jax.jit
def kernel(u_idx, r_idx, v_idx, Eh, rvh):
  eh3 = Eh.reshape(NBLK, 128)
  rvh_flat = rvh.reshape(NUM_REL * DIM)
  u1 = u_idx.astype(jnp.int32)
  r1 = r_idx.astype(jnp.int32)
  v1 = v_idx.astype(jnp.int32)

  mesh = plsc.VectorSubcoreMesh(core_axis_name="c", subcore_axis_name="s")
  run = pl.kernel(
      _body,
      out_type=jax.ShapeDtypeStruct((BATCH,), jnp.float32),
      mesh=mesh,
      compiler_params=pltpu.CompilerParams(
          needs_layout_passes=False, use_tc_tiling_on_sc=True),
      scratch_types=dict(
          uidx=pltpu.VMEM((B_PER_W,), jnp.int32),
          ridx=pltpu.VMEM((B_PER_W,), jnp.int32),
          vidx=pltpu.VMEM((B_PER_W,), jnp.int32),
          rbuf=pltpu.VMEM((NUM_REL * DIM,), jnp.float32),
          ku=pltpu.VMEM((2, G), jnp.int32),
          kv=pltpu.VMEM((2, G), jnp.int32),
          du=pltpu.VMEM((2, G, 128), jnp.float32),
          dv=pltpu.VMEM((2, G, 128), jnp.float32),
          ov=pltpu.VMEM((B_PER_W,), jnp.float32),
          sems=pltpu.SemaphoreType.DMA((2,)),
          rsem=pltpu.SemaphoreType.DMA,
      ),
  )
  return run(eh3, rvh_flat, u1, r1, v1)


# speedup vs baseline: 1.0238x; 1.0238x over previous
import jax
import jax.numpy as jnp
from jax import lax
from jax.experimental import pallas as pl
from jax.experimental.pallas import tpu as pltpu
from jax.experimental.pallas import tpu_sc as plsc

NUM_ENT = 1000000
NUM_REL = 1000
DIM = 32
BATCH = 16384

_INFO = plsc.get_sparse_core_info()
NC = _INFO.num_cores
NS = _INFO.num_subcores
NW = NC * NS
B_PER_W = BATCH // NW
G = 16
NG = B_PER_W // G
NBLK = NUM_ENT


def _newton_sqrt(x):
  bits = plsc.bitcast(x, jnp.int32)
  seed = jnp.int32(0x5F3759DF) - lax.shift_right_logical(bits, 1)
  y = plsc.bitcast(seed, jnp.float32)
  half = x * 0.5
  for _ in range(3):
    y = y * (1.5 - half * y * y)
  return x * y


def _body(eh3, rvh_flat, u1, r1, v1, out, uidx, ridx, vidx,
          rbuf, ku, kv, du, dv, ov, sems, rsem):
  wid = lax.axis_index("s") * NC + lax.axis_index("c")
  base = wid * B_PER_W

  rcp = pltpu.async_copy(rvh_flat, rbuf, rsem)
  pltpu.sync_copy(u1.at[pl.ds(base, B_PER_W)], uidx)
  pltpu.sync_copy(v1.at[pl.ds(base, B_PER_W)], vidx)
  pltpu.sync_copy(r1.at[pl.ds(base, B_PER_W)], ridx)

  iota = lax.iota(jnp.int32, 16)

  def _fire(idxv, i0, kst, dst, sem):
    ivec = idxv[pl.ds(i0, G)]
    kst[...] = ivec
    return pltpu.async_copy(eh3.at[kst], dst, sem)

  for g in (0, 1):
    _fire(uidx, g * G, ku.at[g], du.at[g], sems.at[g])
    _fire(vidx, g * G, kv.at[g], dv.at[g], sems.at[g])

  rcp.wait()

  @pl.loop(0, NG)
  def _group(g):
    slot = lax.rem(g, 2)
    i0 = g * G
    pltpu.make_async_copy(eh3.at[ku.at[slot]], du.at[slot],
                          sems.at[slot]).wait()
    pltpu.make_async_copy(eh3.at[kv.at[slot]], dv.at[slot],
                          sems.at[slot]).wait()

    ivu = uidx[pl.ds(i0, G)]
    ivv = vidx[pl.ds(i0, G)]
    ivr = ridx[pl.ds(i0, G)]
    rofs = ivr * jnp.int32(DIM)
    dus = du.at[slot]
    dvs = dv.at[slot]
    acc = jnp.zeros((16,), jnp.float32)
    for d in range(DIM):
      dd = jnp.full((16,), d, jnp.int32)
      uval = plsc.load_gather(dus, [iota, dd])
      vval = plsc.load_gather(dvs, [iota, dd])
      rval = plsc.load_gather(rbuf, [rofs + jnp.int32(d)])
      diff = uval + rval - vval
      acc = acc + diff * diff
    ov[pl.ds(i0, G)] = _newton_sqrt(acc)

    @pl.when(g < NG - 2)
    def _refill():
      i2 = i0 + 2 * G
      _fire(uidx, i2, ku.at[slot], du.at[slot], sems.at[slot])
      _fire(vidx, i2, kv.at[slot], dv.at[slot], sems.at[slot])

  pltpu.sync_copy(ov, out.at[pl.ds(base, B_PER_W)])


@jax.jit
def kernel(u_idx, r_idx, v_idx, Eh, rvh):
  eh3 = jnp.pad(Eh, ((0, 0), (0, 128 - DIM)))
  rvh_flat = rvh.reshape(NUM_REL * DIM)
  u1 = u_idx.astype(jnp.int32)
  r1 = r_idx.astype(jnp.int32)
  v1 = v_idx.astype(jnp.int32)

  mesh = plsc.VectorSubcoreMesh(core_axis_name="c", subcore_axis_name="s")
  run = pl.kernel(
      _body,
      out_type=jax.ShapeDtypeStruct((BATCH,), jnp.float32),
      mesh=mesh,
      compiler_params=pltpu.CompilerParams(
          needs_layout_passes=False, use_tc_tiling_on_sc=True),
      scratch_types=dict(
          uidx=pltpu.VMEM((B_PER_W,), jnp.int32),
          ridx=pltpu.VMEM((B_PER_W,), jnp.int32),
          vidx=pltpu.VMEM((B_PER_W,), jnp.int32),
          rbuf=pltpu.VMEM((NUM_REL * DIM,), jnp.float32),
          ku=pltpu.VMEM((2, G), jnp.int32),
          kv=pltpu.VMEM((2, G), jnp.int32),
          du=pltpu.VMEM((2, G, 128), jnp.float32),
          dv=pltpu.VMEM((2, G, 128), jnp.float32),
          ov=pltpu.VMEM((B_PER_W,), jnp.float32),
          sems=pltpu.SemaphoreType.DMA((2,)),
          rsem=pltpu.SemaphoreType.DMA,
      ),
  )
  return run(eh3, rvh_flat, u1, r1, v1)


# speedup vs baseline: 1.0449x; 1.0206x over previous
import functools

import jax
import jax.numpy as jnp
from jax import lax
from jax.experimental import pallas as pl
from jax.experimental.pallas import tpu as pltpu
from jax.experimental.pallas import tpu_sc as plsc

NUM_ENT = 1000000
NUM_REL = 1000
DIM = 32
BATCH = 16384

_INFO = plsc.get_sparse_core_info()
NC = _INFO.num_cores
NS = _INFO.num_subcores
NW = NC * NS
B_PER_W = BATCH // NW
CHUNK = 128
NCHUNK = B_PER_W // CHUNK


def _newton_sqrt(x):
  bits = plsc.bitcast(x, jnp.int32)
  seed = jnp.int32(0x5F3759DF) - lax.shift_right_logical(bits, 1)
  y = plsc.bitcast(seed, jnp.float32)
  half = x * 0.5
  for _ in range(3):
    y = y * (1.5 - half * y * y)
  return x * y


def _body(eh, rvh, u2, r2, v2, out, uidx, ridx, vidx, urows, rrows, vrows,
          ssq, sem):
  wid = lax.axis_index("s") * NC + lax.axis_index("c")
  base = wid * B_PER_W

  pltpu.sync_copy(u2.at[pl.ds(wid * NCHUNK, NCHUNK)], uidx)
  pltpu.sync_copy(r2.at[pl.ds(wid * NCHUNK, NCHUNK)], ridx)
  pltpu.sync_copy(v2.at[pl.ds(wid * NCHUNK, NCHUNK)], vidx)

  copies = []
  for j in range(NCHUNK):
    sl = pl.ds(j * CHUNK, CHUNK)
    copies.append(pltpu.async_copy(eh.at[uidx.at[j]], urows.at[sl], sem))
    copies.append(pltpu.async_copy(eh.at[vidx.at[j]], vrows.at[sl], sem))
    copies.append(pltpu.async_copy(rvh.at[ridx.at[j]], rrows.at[sl], sem))
  for c in copies:
    c.wait()

  iota = lax.iota(jnp.int32, 16)

  @plsc.parallel_loop(0, B_PER_W // 16)
  def _grp(g):
    gbase = g * 16
    res = jnp.zeros((16,), jnp.float32)
    for j in range(16):
      i = gbase + j
      u0 = urows[i, pl.ds(0, 16)]
      u1 = urows[i, pl.ds(16, 16)]
      r0 = rrows[i, pl.ds(0, 16)]
      r1 = rrows[i, pl.ds(16, 16)]
      v0 = vrows[i, pl.ds(0, 16)]
      v1 = vrows[i, pl.ds(16, 16)]
      d0 = u0 + r0 - v0
      d1 = u1 + r1 - v1
      h = d0 * d0 + d1 * d1
      res = jnp.where(iota == j, plsc.cumsum(h)[15], res)
    ssq[pl.ds(gbase, 16)] = _newton_sqrt(res)

  pltpu.sync_copy(ssq, out.at[pl.ds(base, B_PER_W)])


@jax.jit
def kernel(u_idx, r_idx, v_idx, Eh, rvh):
  u2 = u_idx.reshape(NW * NCHUNK, CHUNK).astype(jnp.int32)
  r2 = r_idx.reshape(NW * NCHUNK, CHUNK).astype(jnp.int32)
  v2 = v_idx.reshape(NW * NCHUNK, CHUNK).astype(jnp.int32)

  mesh = plsc.VectorSubcoreMesh(core_axis_name="c", subcore_axis_name="s")
  run = pl.kernel(
      _body,
      out_type=jax.ShapeDtypeStruct((BATCH,), jnp.float32),
      mesh=mesh,
      compiler_params=pltpu.CompilerParams(
          needs_layout_passes=False, use_tc_tiling_on_sc=False),
      scratch_types=dict(
          uidx=pltpu.VMEM((NCHUNK, CHUNK), jnp.int32),
          ridx=pltpu.VMEM((NCHUNK, CHUNK), jnp.int32),
          vidx=pltpu.VMEM((NCHUNK, CHUNK), jnp.int32),
          urows=pltpu.VMEM((B_PER_W, DIM), jnp.float32),
          rrows=pltpu.VMEM((B_PER_W, DIM), jnp.float32),
          vrows=pltpu.VMEM((B_PER_W, DIM), jnp.float32),
          ssq=pltpu.VMEM((B_PER_W,), jnp.float32),
          sem=pltpu.SemaphoreType.DMA,
      ),
  )
  return run(Eh, rvh, u2, r2, v2)


# speedup vs baseline: 1.0453x; 1.0003x over previous
import jax
import jax.numpy as jnp
from jax import lax
from jax.experimental import pallas as pl
from jax.experimental.pallas import tpu as pltpu
from jax.experimental.pallas import tpu_sc as plsc

NUM_ENT = 1000000
NUM_REL = 1000
DIM = 32
BATCH = 16384

_INFO = plsc.get_sparse_core_info()
NC = _INFO.num_cores
NS = _INFO.num_subcores
NW = NC * NS
B_PER_W = BATCH // NW
CHUNK = 128
NCHUNK = B_PER_W // CHUNK


def _newton_sqrt(x):
  bits = plsc.bitcast(x, jnp.int32)
  seed = jnp.int32(0x5F3759DF) - lax.shift_right_logical(bits, 1)
  y = plsc.bitcast(seed, jnp.float32)
  half = x * 0.5
  for _ in range(3):
    y = y * (1.5 - half * y * y)
  return x * y


def _body(eh, rvh, u2, r2, v2, out, uidx, ridx, vidx, urows, rrows, vrows,
          ssq, sem):
  wid = lax.axis_index("s") * NC + lax.axis_index("c")
  base = wid * B_PER_W

  pltpu.sync_copy(u2.at[pl.ds(wid * NCHUNK, NCHUNK)], uidx)
  pltpu.sync_copy(r2.at[pl.ds(wid * NCHUNK, NCHUNK)], ridx)
  pltpu.sync_copy(v2.at[pl.ds(wid * NCHUNK, NCHUNK)], vidx)

  copies = []
  for j in range(NCHUNK):
    sl = pl.ds(j * CHUNK, CHUNK)
    copies.append(pltpu.async_copy(eh.at[uidx.at[j]], urows.at[sl], sem))
    copies.append(pltpu.async_copy(eh.at[vidx.at[j]], vrows.at[sl], sem))
    copies.append(pltpu.async_copy(rvh.at[ridx.at[j]], rrows.at[sl], sem))
  for c in copies:
    c.wait()

  iota = lax.iota(jnp.int32, 16)

  @plsc.parallel_loop(0, B_PER_W // 16)
  def _grp(g):
    gbase = g * 16
    res = jnp.zeros((16,), jnp.float32)
    for j in range(16):
      i = gbase + j
      u0 = urows[i, pl.ds(0, 16)]
      u1 = urows[i, pl.ds(16, 16)]
      r0 = rrows[i, pl.ds(0, 16)]
      r1 = rrows[i, pl.ds(16, 16)]
      v0 = vrows[i, pl.ds(0, 16)]
      v1 = vrows[i, pl.ds(16, 16)]
      d0 = u0 + r0 - v0
      d1 = u1 + r1 - v1
      h = d0 * d0 + d1 * d1
      res = jnp.where(iota == j, plsc.cumsum(h)[15], res)
    ssq[pl.ds(gbase, 16)] = _newton_sqrt(res)

  pltpu.sync_copy(ssq, out.at[pl.ds(base, B_PER_W)])


@jax.jit
def kernel(u_idx, r_idx, v_idx, Eh, rvh):
  u2 = u_idx.reshape(NW * NCHUNK, CHUNK).astype(jnp.int32)
  r2 = r_idx.reshape(NW * NCHUNK, CHUNK).astype(jnp.int32)
  v2 = v_idx.reshape(NW * NCHUNK, CHUNK).astype(jnp.int32)

  mesh = plsc.VectorSubcoreMesh(core_axis_name="c", subcore_axis_name="s")
  run = pl.kernel(
      _body,
      out_type=jax.ShapeDtypeStruct((BATCH,), jnp.float32),
      mesh=mesh,
      compiler_params=pltpu.CompilerParams(
          needs_layout_passes=False, use_tc_tiling_on_sc=False),
      scratch_types=dict(
          uidx=pltpu.VMEM((NCHUNK, CHUNK), jnp.int32),
          ridx=pltpu.VMEM((NCHUNK, CHUNK), jnp.int32),
          vidx=pltpu.VMEM((NCHUNK, CHUNK), jnp.int32),
          urows=pltpu.VMEM((B_PER_W, DIM), jnp.float32),
          rrows=pltpu.VMEM((B_PER_W, DIM), jnp.float32),
          vrows=pltpu.VMEM((B_PER_W, DIM), jnp.float32),
          ssq=pltpu.VMEM((B_PER_W,), jnp.float32),
          sem=pltpu.SemaphoreType.DMA,
      ),
  )
  return run(Eh, rvh, u2, r2, v2)
